# baseline (speedup 1.0000x reference)
import functools

import jax
import jax.numpy as jnp
from jax import lax
from jax.experimental import pallas as pl
from jax.experimental.pallas import tpu as pltpu
from jax.experimental.pallas import tpu_sc as plsc

NUM_CHARS = 1000
EMB_DIM = 64


_VPAD = 1024


def _mm_body(emb_ref, w_ref, b_ref, m_ref):
    m_ref[...] = lax.dot_general(
        emb_ref[...], w_ref[...],
        dimension_numbers=(((1,), (1,)), ((), ())),
        preferred_element_type=jnp.float32,
    ) + b_ref[...]


def _make_table(emb_table, W, b):
    w_pad = jnp.zeros((_VPAD, EMB_DIM), jnp.float32).at[:NUM_CHARS].set(W)
    b_pad = jnp.zeros((1, _VPAD), jnp.float32).at[0, :NUM_CHARS].set(b)
    return pl.pallas_call(
        _mm_body,
        out_shape=jax.ShapeDtypeStruct((NUM_CHARS, _VPAD), jnp.float32),
    )(emb_table, w_pad, b_pad)


_NC, _NS = 2, 16
_NW = _NC * _NS

_B = 4096 * 20
_CH = 80
_BPW = _B // _NW
_NCHUNK = _BPW // _CH


def _gather_body(m_hbm, idx_hbm, out_hbm, idx_v, rows_v, sem):
    wid = lax.axis_index("s") * _NC + lax.axis_index("c")
    base = wid * _BPW
    pltpu.sync_copy(idx_hbm.at[pl.ds(base, _BPW)], idx_v)

    def chunk(i, carry):
        off = i * _CH
        pltpu.async_copy(m_hbm.at[idx_v.at[pl.ds(off, _CH)]], rows_v, sem).wait()
        pltpu.sync_copy(rows_v.at[:, pl.ds(0, NUM_CHARS)],
                        out_hbm.at[pl.ds(base + off, _CH)])
        return carry

    lax.fori_loop(0, _NCHUNK, chunk, 0)


@functools.lru_cache(maxsize=1)
def _gather_fn():
    return pl.kernel(
        _gather_body,
        mesh=plsc.VectorSubcoreMesh(core_axis_name="c", subcore_axis_name="s"),
        out_type=jax.ShapeDtypeStruct((_B, NUM_CHARS), jnp.float32),
        scratch_types=[
            pltpu.VMEM((_BPW,), jnp.int32),
            pltpu.VMEM((_CH, _VPAD), jnp.float32),
            pltpu.SemaphoreType.DMA,
        ],
        compiler_params=pltpu.CompilerParams(use_tc_tiling_on_sc=False),
    )


def kernel(x, emb_table, W, b):
    batch, seq = x.shape
    m = _make_table(emb_table, W, b)
    out = _gather_fn()(m, x.reshape(-1).astype(jnp.int32))
    return out.reshape(batch, seq, NUM_CHARS)
